# baseline (speedup 1.0000x reference)
import functools

import jax
import jax.numpy as jnp
from jax import lax
from jax.experimental import pallas as pl
from jax.experimental.pallas import tpu as pltpu
from jax.experimental.pallas import tpu_sc as plsc

B = 16384
F = 26
D = 16
P = (F * (F - 1)) // 2
W = D // 2
XW = F * W
XP = XW + 1

NC = 2
NS = 16
NW = NC * NS

KSPLIT = 2
BK = B // KSPLIT
ROWS_PER_WORKER = BK // NW
CHUNK = 128
GROUPS = CHUNK // 16
NCHUNK = ROWS_PER_WORKER // CHUNK

XLEN = CHUNK * XP - (F - 1) * W
PLEN = CHUNK * XP - (CHUNK - 1) * XW - (F // 2 - 1) * D


def _body(x_hbm, out_hbm, xf, xp, ob):
    wid = lax.axis_index("s") * NC + lax.axis_index("c")
    base = wid * ROWS_PER_WORKER
    iota = lax.iota(jnp.int32, 16)

    def chunk_body(c, _):
        r0 = base + c * CHUNK
        pltpu.sync_copy(x_hbm.at[pl.ds(r0, CHUNK), :], xf)

        def pack_row(r, _):
            rv = iota + r
            for fp in range(F // 2):
                pk = plsc.bitcast(xf[r, pl.ds(fp * 2 * D, 2 * D)], jnp.int32)
                poff = pl.multiple_of(r * XW + fp * D, D)
                plsc.store_scatter(xp.at[pl.ds(poff, PLEN)], [rv], pk)
            return 0

        lax.fori_loop(0, CHUNK, pack_row, 0)

        for g in range(GROUPS):
            rows = g * 16 + iota
            rvx = rows * XP
            rvxw = [rvx + w for w in range(W)]
            rowv = g * 16 + iota

            def ld(fld, w):
                v = plsc.load_gather(
                    xp.at[pl.ds(pl.multiple_of(fld * W, W), XLEN)], [rvxw[w]]
                )
                return plsc.bitcast(v, jnp.bfloat16)

            def st(p, v):
                a, b = plsc.unpack(
                    v,
                    format=plsc.PackFormat.INTERLEAVED,
                    preferred_element_type=jnp.float32,
                )
                pv = jnp.full((16,), 0, jnp.int32) + p
                plsc.store_scatter(ob, [rowv, pv], a + b)

            def i_body(h, _):
                i = 2 * h
                ra = [ld(i, w) for w in range(W)]
                rb = [ld(i + 1, w) for w in range(W)]
                acc = ra[0] * rb[0]
                accq = ra[1] * rb[1]
                for w in range(2, W, 2):
                    acc = acc + ra[w] * rb[w]
                    accq = accq + ra[w + 1] * rb[w + 1]
                pa = 25 * i - (i * (i - 1)) // 2
                st(pa, acc + accq)
                pb = pa + 24 - i

                def col_accs(j):
                    c0 = ld(j, 0)
                    c1 = ld(j, 1)
                    acc0 = ra[0] * c0
                    acc1 = rb[0] * c0
                    acc2 = ra[1] * c1
                    acc3 = rb[1] * c1
                    for w in range(2, W, 2):
                        cw = ld(j, w)
                        cv = ld(j, w + 1)
                        acc0 = acc0 + ra[w] * cw
                        acc1 = acc1 + rb[w] * cw
                        acc2 = acc2 + ra[w + 1] * cv
                        acc3 = acc3 + rb[w + 1] * cv
                    return acc0 + acc2, acc1 + acc3

                def j_body(t, _):
                    j = i + 2 + 2 * t
                    s0, s1 = col_accs(j)
                    s2, s3 = col_accs(j + 1)
                    st(pa + (j - i - 1), s0)
                    st(pb + (j - i - 1), s1)
                    st(pa + (j - i), s2)
                    st(pb + (j - i), s3)
                    return 0

                lax.fori_loop(0, (F - 2 - i) // 2, j_body, 0)
                return 0

            lax.fori_loop(0, F // 2, i_body, 0)
        pltpu.sync_copy(ob, out_hbm.at[pl.ds(r0, CHUNK), :])
        return 0

    lax.fori_loop(0, NCHUNK, chunk_body, 0)


@jax.jit
def _run(x16):
    mesh = plsc.VectorSubcoreMesh(
        core_axis_name="c", subcore_axis_name="s", num_cores=NC, num_subcores=NS
    )
    f = pl.kernel(
        _body,
        out_type=jax.ShapeDtypeStruct((BK, P), jnp.float32),
        mesh=mesh,
        scratch_types=[
            pltpu.VMEM((CHUNK, F * D), jnp.bfloat16),
            pltpu.VMEM((CHUNK * XP,), jnp.int32),
            pltpu.VMEM((CHUNK, P), jnp.float32),
        ],
        compiler_params=pltpu.CompilerParams(
            needs_layout_passes=False, use_tc_tiling_on_sc=False
        ),
    )
    outs = []
    for k in range(KSPLIT):
        xk = lax.slice_in_dim(x16, k * BK, (k + 1) * BK, axis=0)
        outs.append(f(xk))
    return jnp.concatenate(outs, axis=0)


def kernel(x):
    x16 = x.reshape(B, F * D).astype(jnp.bfloat16)
    return _run(x16)


# speedup vs baseline: 1.0713x; 1.0713x over previous
import jax
import jax.numpy as jnp
from jax import lax
from jax.experimental import pallas as pl
from jax.experimental.pallas import tpu as pltpu
from jax.experimental.pallas import tpu_sc as plsc

B = 16384
F = 26
D = 16
P = (F * (F - 1)) // 2
W = D // 2
XW = F * W
XP = XW + 1

NC = 2
NS = 16
NW = NC * NS

ROWS_PER_WORKER = B // NW
CHUNK = 128
GROUPS = CHUNK // 16
NCHUNK = ROWS_PER_WORKER // CHUNK

XLEN = CHUNK * XP - (F - 1) * W
PLEN = CHUNK * XP - (CHUNK - 1) * XW - (F // 2 - 1) * D


def _body(x_hbm, out_hbm, xf, xp, ob):
    wid = lax.axis_index("s") * NC + lax.axis_index("c")
    base = wid * ROWS_PER_WORKER
    iota = lax.iota(jnp.int32, 16)

    def chunk_body(c, _):
        r0 = base + c * CHUNK
        pltpu.sync_copy(x_hbm.at[pl.ds(r0, CHUNK), :], xf)

        def pack_row(r, _):
            rv = iota + r
            for fp in range(F // 2):
                pk = plsc.bitcast(xf[r, pl.ds(fp * 2 * D, 2 * D)], jnp.int32)
                poff = pl.multiple_of(r * XW + fp * D, D)
                plsc.store_scatter(xp.at[pl.ds(poff, PLEN)], [rv], pk)
            return 0

        lax.fori_loop(0, CHUNK, pack_row, 0)

        for g in range(GROUPS):
            rows = g * 16 + iota
            rvx = rows * XP
            rvxw = [rvx + w for w in range(W)]
            rowv = g * 16 + iota

            def ld(fld, w):
                v = plsc.load_gather(
                    xp.at[pl.ds(pl.multiple_of(fld * W, W), XLEN)], [rvxw[w]]
                )
                return plsc.bitcast(v, jnp.bfloat16)

            def st(p, v):
                a, b = plsc.unpack(
                    v,
                    format=plsc.PackFormat.INTERLEAVED,
                    preferred_element_type=jnp.float32,
                )
                pv = jnp.full((16,), 0, jnp.int32) + p
                plsc.store_scatter(ob, [rowv, pv], a + b)

            def i_body(h, _):
                i = 2 * h
                ra = [ld(i, w) for w in range(W)]
                rb = [ld(i + 1, w) for w in range(W)]
                acc = ra[0] * rb[0]
                accq = ra[1] * rb[1]
                for w in range(2, W, 2):
                    acc = acc + ra[w] * rb[w]
                    accq = accq + ra[w + 1] * rb[w + 1]
                pa = 25 * i - (i * (i - 1)) // 2
                st(pa, acc + accq)
                pb = pa + 24 - i

                def col_accs(j):
                    c0 = ld(j, 0)
                    c1 = ld(j, 1)
                    acc0 = ra[0] * c0
                    acc1 = rb[0] * c0
                    acc2 = ra[1] * c1
                    acc3 = rb[1] * c1
                    for w in range(2, W, 2):
                        cw = ld(j, w)
                        cv = ld(j, w + 1)
                        acc0 = acc0 + ra[w] * cw
                        acc1 = acc1 + rb[w] * cw
                        acc2 = acc2 + ra[w + 1] * cv
                        acc3 = acc3 + rb[w + 1] * cv
                    return acc0 + acc2, acc1 + acc3

                def j_body(t, _):
                    j = i + 2 + 2 * t
                    s0, s1 = col_accs(j)
                    s2, s3 = col_accs(j + 1)
                    st(pa + (j - i - 1), s0)
                    st(pb + (j - i - 1), s1)
                    st(pa + (j - i), s2)
                    st(pb + (j - i), s3)
                    return 0

                lax.fori_loop(0, (F - 2 - i) // 2, j_body, 0)
                return 0

            lax.fori_loop(0, F // 2, i_body, 0)
        pltpu.sync_copy(ob, out_hbm.at[pl.ds(r0, CHUNK), :])
        return 0

    lax.fori_loop(0, NCHUNK, chunk_body, 0)


@jax.jit
def _run(x16):
    mesh = plsc.VectorSubcoreMesh(
        core_axis_name="c", subcore_axis_name="s", num_cores=NC, num_subcores=NS
    )
    f = pl.kernel(
        _body,
        out_type=jax.ShapeDtypeStruct((B, P), jnp.float32),
        mesh=mesh,
        scratch_types=[
            pltpu.VMEM((CHUNK, F * D), jnp.bfloat16),
            pltpu.VMEM((CHUNK * XP,), jnp.int32),
            pltpu.VMEM((CHUNK, P), jnp.float32),
        ],
        compiler_params=pltpu.CompilerParams(
            needs_layout_passes=False, use_tc_tiling_on_sc=False
        ),
    )
    return f(x16)


def kernel(x):
    x16 = x.reshape(B, F * D).astype(jnp.bfloat16)
    return _run(x16)
